# baseline (speedup 1.0000x reference)
import functools

import jax
import jax.numpy as jnp
from jax import lax
from jax.experimental import pallas as pl
from jax.experimental.pallas import tpu as pltpu
from jax.experimental.pallas import tpu_sc as plsc

_V = 1000000
_D = 64
_B, _L = 4096, 200
_N = _B * _L

_NC = 2
_NS = 16
_NW = _NC * _NS
_PER_W = _N // _NW
_C = 128
_NCHUNK = _PER_W // _C

_mesh = plsc.VectorSubcoreMesh(core_axis_name="c", subcore_axis_name="s")


@functools.partial(
    pl.kernel,
    out_type=jax.ShapeDtypeStruct((_B, _L, _D), jnp.float32),
    mesh=_mesh,
    compiler_params=pltpu.CompilerParams(needs_layout_passes=False),
    scratch_types=[
        pltpu.VMEM((_PER_W,), jnp.int32),
        pltpu.VMEM((2, _C, 2 * _D), jnp.float32),
        pltpu.VMEM((2, _C, _D), jnp.float32),
        pltpu.SemaphoreType.DMA,
        pltpu.SemaphoreType.DMA,
        pltpu.SemaphoreType.DMA,
        pltpu.SemaphoreType.DMA,
    ],
)
def _emb_lookup(tall, idx_hbm, out_hbm, idx_v, rows_v, rows_c,
                sg0, sg1, sw0, sw1):
    out2 = out_hbm.reshape(_N, _D)
    wid = lax.axis_index("s") * _NC + lax.axis_index("c")
    base = wid * _PER_W
    gsems = (sg0, sg1)
    wsems = (sw0, sw1)

    pltpu.sync_copy(idx_hbm.at[pl.ds(base, _PER_W)], idx_v)

    def gather_copy(i, b):
        return pltpu.make_async_copy(
            tall.at[idx_v.at[pl.ds(i * _C, _C)]], rows_v.at[b], gsems[b])

    def start_gather(i, b):
        pltpu.async_copy(
            tall.at[idx_v.at[pl.ds(i * _C, _C)]], rows_v.at[b], gsems[b])

    def compact(b):
        for r in range(_C):
            for k in range(_D // 16):
                rows_c[b, r, pl.ds(16 * k, 16)] = (
                    rows_v[b, r, pl.ds(16 * k, 16)])

    def start_write(i, b):
        pltpu.async_copy(
            rows_c.at[b], out2.at[pl.ds(base + i * _C, _C)], wsems[b])

    def wait_write(i, b):
        pltpu.make_async_copy(
            rows_c.at[b], out2.at[pl.ds(base + i * _C, _C)], wsems[b]).wait()

    start_gather(0, 0)

    def pair_body(io, carry):
        for b in range(2):
            i = 2 * io + b

            @pl.when(i + 1 < _NCHUNK)
            def _():
                start_gather(i + 1, 1 - b)

            gather_copy(i, b).wait()

            @pl.when(i >= 2)
            def _():
                wait_write(i - 2, b)

            compact(b)
            start_write(i, b)
        return carry

    lax.fori_loop(0, _NCHUNK // 2, pair_body, 0)
    wait_write(_NCHUNK - 2, 0)
    wait_write(_NCHUNK - 1, 1)


def kernel(y, table):
    packed_even = table.reshape(_V // 2, 2 * _D)
    packed_odd = jnp.concatenate([table[1:], table[:1]], axis=0).reshape(
        _V // 2, 2 * _D)
    tall = jnp.concatenate([packed_even, packed_odd], axis=0)
    yf = y.reshape(_N)
    idxg = (yf >> 1) + (yf & 1) * (_V // 2)
    return _emb_lookup(tall, idxg)


# speedup vs baseline: 1.7235x; 1.7235x over previous
import functools

import jax
import jax.numpy as jnp
from jax import lax
from jax.experimental import pallas as pl
from jax.experimental.pallas import tpu as pltpu
from jax.experimental.pallas import tpu_sc as plsc

_V = 1000000
_D = 64
_B, _L = 4096, 200
_N = _B * _L

_NC = 2
_NS = 16
_NW = _NC * _NS
_PER_W = _N // _NW
_C = 128
_NCHUNK = _PER_W // _C

_mesh = plsc.VectorSubcoreMesh(core_axis_name="c", subcore_axis_name="s")


@functools.partial(
    pl.kernel,
    out_type=jax.ShapeDtypeStruct((_B, _L, _D), jnp.float32),
    mesh=_mesh,
    compiler_params=pltpu.CompilerParams(needs_layout_passes=False),
    scratch_types=[
        pltpu.VMEM((_PER_W,), jnp.int32),
        pltpu.VMEM((2, _C, 16), jnp.int32),
        pltpu.VMEM((2, _C, 2 * _D), jnp.float32),
        pltpu.VMEM((2, _C, _D), jnp.float32),
        pltpu.SemaphoreType.DMA,
        pltpu.SemaphoreType.DMA,
        pltpu.SemaphoreType.DMA,
        pltpu.SemaphoreType.DMA,
        pltpu.SemaphoreType.DMA,
        pltpu.SemaphoreType.DMA,
    ],
)
def _emb_lookup(packed, idxj_hbm, pm_hbm, out_hbm, idxj_v, pm_v, rows_v,
                rows_c, sg0, sg1, sm0, sm1, sw0, sw1):
    out2 = out_hbm.reshape(_N, _D)
    pm3 = pm_hbm.reshape(_N, 16)
    wid = lax.axis_index("s") * _NC + lax.axis_index("c")
    base = wid * _PER_W
    gsems = (sg0, sg1)
    msems = (sm0, sm1)
    wsems = (sw0, sw1)
    zero = jnp.zeros((16,), jnp.int32)

    pltpu.sync_copy(idxj_hbm.at[pl.ds(base, _PER_W)], idxj_v)

    def start_gather(i, b):
        pltpu.async_copy(
            packed.at[idxj_v.at[pl.ds(i * _C, _C)]], rows_v.at[b], gsems[b])
        pltpu.async_copy(
            pm3.at[pl.ds(base + i * _C, _C)], pm_v.at[b], msems[b])

    def wait_gather(i, b):
        pltpu.make_async_copy(
            packed.at[idxj_v.at[pl.ds(i * _C, _C)]], rows_v.at[b],
            gsems[b]).wait()
        pltpu.make_async_copy(
            pm3.at[pl.ds(base + i * _C, _C)], pm_v.at[b], msems[b]).wait()

    def compact(b):
        def blk_body(rb, carry):
            for rr in range(8):
                r = rb * 8 + rr
                m = pm_v[b, r, pl.ds(0, 16)] != zero
                for k in range(_D // 16):
                    lo = rows_v[b, r, pl.ds(16 * k, 16)]
                    hi = rows_v[b, r, pl.ds(_D + 16 * k, 16)]
                    rows_c[b, r, pl.ds(16 * k, 16)] = jnp.where(m, hi, lo)
            return carry

        lax.fori_loop(0, _C // 8, blk_body, 0)

    def start_write(i, b):
        pltpu.async_copy(
            rows_c.at[b], out2.at[pl.ds(base + i * _C, _C)], wsems[b])

    def wait_write(i, b):
        pltpu.make_async_copy(
            rows_c.at[b], out2.at[pl.ds(base + i * _C, _C)], wsems[b]).wait()

    start_gather(0, 0)

    def pair_body(io, carry):
        for b in range(2):
            i = 2 * io + b

            @pl.when(i + 1 < _NCHUNK)
            def _():
                start_gather(i + 1, 1 - b)

            wait_gather(i, b)

            @pl.when(i >= 2)
            def _():
                wait_write(i - 2, b)

            compact(b)
            start_write(i, b)
        return carry

    lax.fori_loop(0, _NCHUNK // 2, pair_body, 0)
    wait_write(_NCHUNK - 2, 0)
    wait_write(_NCHUNK - 1, 1)


def kernel(y, table):
    packed = table.reshape(_V // 2, 2 * _D)
    yf = y.reshape(_N)
    idxj = yf >> 1
    pm = jnp.broadcast_to((yf & 1)[:, None], (_N, 16))
    return _emb_lookup(packed, idxj, pm)
